# baseline (speedup 1.0000x reference)
import jax
import jax.numpy as jnp
from jax import lax
from jax.experimental import pallas as pl
from jax.experimental.pallas import tpu as pltpu
from jax.experimental.pallas import tpu_sc as plsc

BATCH = 4096
SEQ = 200
DIM = 64
N = BATCH * SEQ

NUM_CORES = 2
NUM_SUBCORES = 16
NW = NUM_CORES * NUM_SUBCORES
ROWS_PER_W = N // NW

GB = 50
KSUB = 8
CHUNK = GB * KSUB
G = ROWS_PER_W // CHUNK

NBUF = 4
XROWS = N // GB


def _body(xf_hbm, table_hbm, tmpl_hbm, out_hbm,
          iv0, iv1, iv2, iv3, b0, b1, b2, b3,
          si0, si1, si2, si3, sg0, sg1, sg2, sg3, ss0, ss1, ss2, ss3):
    idx_v = [iv0, iv1, iv2, iv3]
    buf = [b0, b1, b2, b3]
    sem_in = [si0, si1, si2, si3]
    sem_g = [sg0, sg1, sg2, sg3]
    sem_s = [ss0, ss1, ss2, ss3]

    wid = lax.axis_index("s") * NUM_CORES + lax.axis_index("c")
    base0 = wid * ROWS_PER_W

    def in_copies(g, b):
        idx_off = pl.multiple_of(wid * (ROWS_PER_W // GB) + g * KSUB, KSUB)
        return (
            pltpu.make_async_copy(
                xf_hbm.at[pl.ds(idx_off, KSUB)], idx_v[b], sem_in[b]
            ),
            pltpu.make_async_copy(tmpl_hbm.at[pl.ds(0, CHUNK)], buf[b], sem_in[b]),
        )

    def gather_copies(b):
        return [
            pltpu.make_async_copy(
                table_hbm.at[idx_v[b].at[k]],
                buf[b].at[pl.ds(k * GB, GB)],
                sem_g[b],
            )
            for k in range(KSUB)
        ]

    def out_copy(g, b):
        base = pl.multiple_of(base0 + g * CHUNK, CHUNK)
        return pltpu.make_async_copy(buf[b], out_hbm.at[pl.ds(base, CHUNK)], sem_s[b])

    def step(g, b, bn, steady):
        for c in in_copies(g, b):
            c.wait()
        gs = gather_copies(b)
        for c in gs:
            c.start(add=True)
        if steady:
            out_copy(g - (NBUF - 1), bn).wait()
        if bn is not None:
            for c in in_copies(g + 1, bn):
                c.start()
        for c in gs:
            c.wait()
        out_copy(g, b).start()

    for c in in_copies(0, 0):
        c.start()
    step(0, 0, 1, False)
    step(1, 1, 2, False)
    step(2, 2, 3, False)

    def ring(blk, carry):
        g0 = 3 + blk * NBUF
        step(g0 + 0, 3, 0, True)
        step(g0 + 1, 0, 1, True)
        step(g0 + 2, 1, 2, True)
        step(g0 + 3, 2, 3, True)
        return carry

    lax.fori_loop(0, (G - 4) // NBUF, ring, 0)

    step(G - 1, (G - 1) % NBUF, None, False)
    for g in range(G - NBUF, G):
        out_copy(g, g % NBUF).wait()


@jax.jit
def _run(xf, table, tmpl):
    mesh = plsc.VectorSubcoreMesh(core_axis_name="c", subcore_axis_name="s")
    f = pl.kernel(
        _body,
        out_type=jax.ShapeDtypeStruct((N, DIM), jnp.float32),
        mesh=mesh,
        compiler_params=pltpu.CompilerParams(use_tc_tiling_on_sc=False),
        scratch_types=(
            [pltpu.VMEM((KSUB, GB), jnp.int32) for _ in range(NBUF)]
            + [pltpu.VMEM((CHUNK, DIM), jnp.float32) for _ in range(NBUF)]
            + [pltpu.SemaphoreType.DMA for _ in range(3 * NBUF)]
        ),
    )
    return f(xf, table, tmpl)


def kernel(x, table, pos_encoding):
    xf = x.reshape(XROWS, GB).astype(jnp.int32)
    tmpl = jnp.tile(pos_encoding[:SEQ], (CHUNK // SEQ, 1))
    out = _run(xf, table, tmpl)
    return out.reshape(BATCH, SEQ, DIM)


# speedup vs baseline: 1.2265x; 1.2265x over previous
import jax
import jax.numpy as jnp
from jax import lax
from jax.experimental import pallas as pl
from jax.experimental.pallas import tpu as pltpu
from jax.experimental.pallas import tpu_sc as plsc

BATCH = 4096
SEQ = 200
DIM = 64
N = BATCH * SEQ

NUM_CORES = 2
NUM_SUBCORES = 16
NW = NUM_CORES * NUM_SUBCORES
ROWS_PER_W = N // NW

GB = 100
KSUB = 16
CHUNK = GB * KSUB
G = ROWS_PER_W // CHUNK

XROWS = N // GB


def _body(xf_hbm, table_hbm, tmpl_hbm, out_hbm, idx_v, buf, sem):
    wid = lax.axis_index("s") * NUM_CORES + lax.axis_index("c")
    base0 = wid * ROWS_PER_W

    def chunk_body(g, carry):
        base = pl.multiple_of(base0 + g * CHUNK, CHUNK)
        idx_off = pl.multiple_of(wid * (ROWS_PER_W // GB) + g * KSUB, KSUB)
        pltpu.sync_copy(xf_hbm.at[pl.ds(idx_off, KSUB)], idx_v)
        pltpu.sync_copy(tmpl_hbm.at[pl.ds(0, CHUNK)], buf)
        copies = [
            pltpu.async_copy(
                table_hbm.at[idx_v.at[k]],
                buf.at[pl.ds(k * GB, GB)],
                sem,
                add=True,
            )
            for k in range(KSUB)
        ]
        for c in copies:
            c.wait()
        pltpu.sync_copy(buf, out_hbm.at[pl.ds(base, CHUNK)])
        return carry

    lax.fori_loop(0, G, chunk_body, 0)


@jax.jit
def _run(xf, table, tmpl):
    mesh = plsc.VectorSubcoreMesh(core_axis_name="c", subcore_axis_name="s")
    f = pl.kernel(
        _body,
        out_type=jax.ShapeDtypeStruct((N, DIM), jnp.float32),
        mesh=mesh,
        compiler_params=pltpu.CompilerParams(use_tc_tiling_on_sc=False),
        scratch_types=[
            pltpu.VMEM((KSUB, GB), jnp.int32),
            pltpu.VMEM((CHUNK, DIM), jnp.float32),
            pltpu.SemaphoreType.DMA,
        ],
    )
    return f(xf, table, tmpl)


def kernel(x, table, pos_encoding):
    xf = x.reshape(XROWS, GB).astype(jnp.int32)
    tmpl = jnp.tile(pos_encoding[:SEQ], (CHUNK // SEQ, 1))
    out = _run(xf, table, tmpl)
    return out.reshape(BATCH, SEQ, DIM)


# speedup vs baseline: 1.2343x; 1.0063x over previous
import jax
import jax.numpy as jnp
from jax import lax
from jax.experimental import pallas as pl
from jax.experimental.pallas import tpu as pltpu
from jax.experimental.pallas import tpu_sc as plsc

BATCH = 4096
SEQ = 200
DIM = 64
N = BATCH * SEQ

NUM_CORES = 2
NUM_SUBCORES = 16
NW = NUM_CORES * NUM_SUBCORES
ROWS_PER_W = N // NW

CHUNK = 1600
G = ROWS_PER_W // CHUNK


def _body(xf_hbm, table_hbm, tmpl_hbm, out_hbm, idx_v, buf, sem):
    wid = lax.axis_index("s") * NUM_CORES + lax.axis_index("c")
    base0 = wid * ROWS_PER_W

    def chunk_body(g, carry):
        base = pl.multiple_of(base0 + g * CHUNK, CHUNK)
        pltpu.sync_copy(xf_hbm.at[pl.ds(base, CHUNK)], idx_v)
        pltpu.sync_copy(tmpl_hbm.at[pl.ds(0, CHUNK)], buf)
        pltpu.async_copy(table_hbm.at[idx_v], buf, sem, add=True).wait()
        pltpu.sync_copy(buf, out_hbm.at[pl.ds(base, CHUNK)])
        return carry

    lax.fori_loop(0, G, chunk_body, 0)


@jax.jit
def _run(xf, table, tmpl):
    mesh = plsc.VectorSubcoreMesh(core_axis_name="c", subcore_axis_name="s")
    f = pl.kernel(
        _body,
        out_type=jax.ShapeDtypeStruct((N, DIM), jnp.float32),
        mesh=mesh,
        compiler_params=pltpu.CompilerParams(use_tc_tiling_on_sc=False),
        scratch_types=[
            pltpu.VMEM((CHUNK,), jnp.int32),
            pltpu.VMEM((CHUNK, DIM), jnp.float32),
            pltpu.SemaphoreType.DMA,
        ],
    )
    return f(xf, table, tmpl)


def kernel(x, table, pos_encoding):
    xf = x.reshape(N).astype(jnp.int32)
    tmpl = jnp.tile(pos_encoding[:SEQ], (CHUNK // SEQ, 1))
    out = _run(xf, table, tmpl)
    return out.reshape(BATCH, SEQ, DIM)
